# Initial kernel scaffold; baseline (speedup 1.0000x reference)
import jax
import jax.numpy as jnp
from jax.experimental import pallas as pl


def kernel(embeddings, sp_seg, affs, offs, edges, pair_edge_ids, pair_pix_a, pair_pix_b):
    raise NotImplementedError("write your pallas kernel here")


# speedup vs baseline: 16.2072x; 16.2072x over previous
import functools

import jax
import jax.numpy as jnp
from jax import lax
from jax.experimental import pallas as pl
from jax.experimental.pallas import tpu as pltpu
from jax.experimental.pallas import tpu_sc as plsc

_DELTA_VAR = 0.1
_DELTA_DIST = 0.3
_ALPHA = 1.0
_BETA = 1.0

_C = 128
_D = 16
_HW = 128 * 128

_NT = 16
_L = 16


def _tc_body(seg_ref, emb_ref, aff_ref, hflat_ref, means_ref, intra_ref):
    seg = seg_ref[...]
    emb = emb_ref[...]
    ids = lax.broadcasted_iota(jnp.int32, (_C, _HW), 0)
    oh = (ids == seg).astype(jnp.float32)
    dn = (((1,), (1,)), ((), ()))
    sp_sums = lax.dot_general(oh, emb, dn, preferred_element_type=jnp.float32)
    n = jnp.sum(oh, axis=1, keepdims=True)
    inv_n = 1.0 / jnp.maximum(n, 1.0)
    means = sp_sums * inv_n
    dn0 = (((0,), (0,)), ((), ()))
    mean_t = lax.dot_general(means, oh, dn0, preferred_element_type=jnp.float32)
    dots = jnp.sum(mean_t * emb, axis=0, keepdims=True)
    t = jnp.clip(1.0 - dots - _DELTA_VAR, 0.0, None)
    seg_t = lax.dot_general(oh, t, dn, preferred_element_type=jnp.float32)
    c_t = (jnp.max(seg) + 1).astype(jnp.float32)
    intra_ref[...] = jnp.reshape(jnp.sum(seg_t * inv_n) / c_t, (1, 1))

    a0 = aff_ref[0:1, :]
    a1 = aff_ref[1:2, :]
    hraw = 0.5 * (a0 + a1)
    hs = hraw - jnp.min(hraw)
    hflat_ref[...] = hs / (jnp.max(hs) + 1e-6)
    means_ref[...] = means


def _tc_call(seg_row, emb, aff2):
    return pl.pallas_call(
        _tc_body,
        out_shape=(
            jax.ShapeDtypeStruct((1, _HW), jnp.float32),
            jax.ShapeDtypeStruct((_C, _D), jnp.float32),
            jax.ShapeDtypeStruct((1, 1), jnp.float32),
        ),
    )(seg_row, emb, aff2)


def _sc_body(P, E, Ppad, Epad,
             hflat_hbm, spm_hbm, pa_hbm, pb_hbm, peid_hbm, eu_hbm, ev_hbm,
             out_hbm,
             hflat_v, spm_v, pa_v, pb_v, peid_v, vals_v, ones_v,
             eu_v, ev_v, sums_v, cnts_v, part_v, partall_v, out_v,
             sums_sh, cnts_sh, part_sh):
    cid = lax.axis_index("c")
    sid = lax.axis_index("s")
    pchunk = Ppad // _NT
    echunk = Epad // _NT
    n_pgrp = pchunk // _L
    n_egrp = echunk // _L
    n_prow = pchunk // 128

    @pl.when(cid == 0)
    def _():
        pltpu.sync_copy(hflat_hbm, hflat_v)
        pltpu.sync_copy(spm_hbm, spm_v)
        pltpu.sync_copy(pa_hbm.at[sid], pa_v)
        pltpu.sync_copy(pb_hbm.at[sid], pb_v)
        pltpu.sync_copy(peid_hbm.at[sid], peid_v)
        pltpu.sync_copy(eu_hbm.at[sid], eu_v)
        pltpu.sync_copy(ev_hbm.at[sid], ev_v)

        def zero_body(j, _):
            off = pl.multiple_of(j * _L, _L)
            sums_v[pl.ds(off, _L)] = jnp.zeros((_L,), jnp.float32)
            return 0
        lax.fori_loop(0, echunk // _L, zero_body, 0)
        pltpu.sync_copy(sums_v, sums_sh.at[pl.ds(sid * echunk, echunk)])
        pltpu.sync_copy(sums_v, cnts_sh.at[pl.ds(sid * echunk, echunk)])
        plsc.subcore_barrier()

        base_p = sid * pchunk
        lanes = lax.iota(jnp.int32, _L)

        def pair_body(g, _):
            off = pl.multiple_of(g * _L, _L)
            ia = pa_v[pl.ds(off, _L)]
            ib = pb_v[pl.ds(off, _L)]
            ha = plsc.load_gather(hflat_v, [ia])
            hb = plsc.load_gather(hflat_v, [ib])
            val = 0.5 * (ha + hb)
            gidx = base_p + g * _L + lanes
            m = gidx < P
            q = lax.div(g, 8)
            r = lax.rem(g, 8)
            roff = pl.multiple_of(r * _L, _L)
            vals_v[q, pl.ds(roff, _L)] = jnp.where(m, val, 0.0)
            ones_v[q, pl.ds(roff, _L)] = jnp.where(m, 1.0, 0.0)
            return 0
        lax.fori_loop(0, n_pgrp, pair_body, 0)

        def scat_body(j, _):
            idx_row = peid_v.at[j]
            pltpu.sync_copy(vals_v.at[j], sums_sh.at[idx_row], add=True)
            pltpu.sync_copy(ones_v.at[j], cnts_sh.at[idx_row], add=True)
            return 0
        lax.fori_loop(0, n_prow, scat_body, 0)
        plsc.subcore_barrier()

        base_e = sid * echunk
        pltpu.sync_copy(sums_sh.at[pl.ds(base_e, echunk)], sums_v)
        pltpu.sync_copy(cnts_sh.at[pl.ds(base_e, echunk)], cnts_v)

        def edge_body(g, acc):
            off = pl.multiple_of(g * _L, _L)
            u = eu_v[pl.ds(off, _L)]
            v = ev_v[pl.ds(off, _L)]
            s = sums_v[pl.ds(off, _L)]
            c = cnts_v[pl.ds(off, _L)]
            w = s / jnp.maximum(c, 1.0)
            ub = u * _D
            vb = v * _D
            dacc = jnp.zeros((_L,), jnp.float32)
            for dd in range(_D):
                mu = plsc.load_gather(spm_v, [ub + dd])
                mv = plsc.load_gather(spm_v, [vb + dd])
                dacc = dacc + mu * mv
            inter = jnp.clip(_DELTA_DIST - (1.0 - dacc) * w, 0.0, None)
            gidx = base_e + g * _L + lanes
            return acc + jnp.where(gidx < E, inter, 0.0)

        acc = lax.fori_loop(0, n_egrp, edge_body, jnp.zeros((_L,), jnp.float32))
        part_v[...] = acc
        pltpu.sync_copy(part_v, part_sh.at[sid])
        plsc.subcore_barrier()

        @pl.when(sid == 0)
        def _():
            pltpu.sync_copy(part_sh, partall_v)
            tot = jnp.zeros((_L,), jnp.float32)
            for t in range(_NT):
                tot = tot + partall_v[t]
            total = lax.reduce(tot, 0.0, lax.add, (0,))
            out_v[...] = jnp.full((_L,), total * (1.0 / E), jnp.float32)
            pltpu.sync_copy(out_v, out_hbm)


def _sc_call(P, E, Ppad, Epad, hflat, spm, pa3, pb3, peid3, eu2, ev2):
    pchunk = Ppad // _NT
    echunk = Epad // _NT
    n_prow = pchunk // 128
    mesh = plsc.VectorSubcoreMesh(core_axis_name="c", subcore_axis_name="s",
                                  num_cores=2, num_subcores=_NT)
    kern = pl.kernel(
        functools.partial(_sc_body, P, E, Ppad, Epad),
        out_type=jax.ShapeDtypeStruct((_L,), jnp.float32),
        mesh=mesh,
        compiler_params=pltpu.CompilerParams(needs_layout_passes=False),
        scratch_types=[
            pltpu.VMEM((_HW,), jnp.float32),
            pltpu.VMEM((_C * _D,), jnp.float32),
            pltpu.VMEM((pchunk,), jnp.int32),
            pltpu.VMEM((pchunk,), jnp.int32),
            pltpu.VMEM((n_prow, 128), jnp.int32),
            pltpu.VMEM((n_prow, 128), jnp.float32),
            pltpu.VMEM((n_prow, 128), jnp.float32),
            pltpu.VMEM((echunk,), jnp.int32),
            pltpu.VMEM((echunk,), jnp.int32),
            pltpu.VMEM((echunk,), jnp.float32),
            pltpu.VMEM((echunk,), jnp.float32),
            pltpu.VMEM((_L,), jnp.float32),
            pltpu.VMEM((_NT, _L), jnp.float32),
            pltpu.VMEM((_L,), jnp.float32),
            pltpu.VMEM_SHARED((Epad,), jnp.float32),
            pltpu.VMEM_SHARED((Epad,), jnp.float32),
            pltpu.VMEM_SHARED((_NT, _L), jnp.float32),
        ],
    )
    return kern(hflat, spm, pa3, pb3, peid3, eu2, ev2)


def _pad_to(x, n):
    return jnp.concatenate([x, jnp.zeros((n - x.shape[0],), x.dtype)])


def kernel(embeddings, sp_seg, affs, offs, edges, pair_edge_ids,
           pair_pix_a, pair_pix_b):
    del offs
    seg_row = sp_seg.reshape(1, _HW).astype(jnp.int32)
    emb = embeddings.reshape(_D, _HW).astype(jnp.float32)
    aff2 = affs[0, :2].reshape(2, _HW).astype(jnp.float32)

    hflat, means, intra = _tc_call(seg_row, emb, aff2)

    P = pair_pix_a.shape[0]
    E = edges.shape[1]
    Ppad = -(-P // (_NT * 128)) * (_NT * 128)
    Epad = -(-E // (_NT * 128)) * (_NT * 128)
    pchunk = Ppad // _NT
    echunk = Epad // _NT
    pa3 = _pad_to(pair_pix_a.astype(jnp.int32), Ppad).reshape(_NT, pchunk)
    pb3 = _pad_to(pair_pix_b.astype(jnp.int32), Ppad).reshape(_NT, pchunk)
    peid3 = _pad_to(pair_edge_ids.astype(jnp.int32), Ppad).reshape(
        _NT, pchunk // 128, 128)
    eu2 = _pad_to(edges[0].astype(jnp.int32), Epad).reshape(_NT, echunk)
    ev2 = _pad_to(edges[1].astype(jnp.int32), Epad).reshape(_NT, echunk)

    inter_vec = _sc_call(P, E, Ppad, Epad,
                         hflat.reshape(_HW), means.reshape(_C * _D),
                         pa3, pb3, peid3, eu2, ev2)
    loss = _ALPHA * inter_vec[0] + _BETA * intra[0, 0]
    return loss


# speedup vs baseline: 19.1128x; 1.1793x over previous
import functools

import jax
import jax.numpy as jnp
from jax import lax
from jax.experimental import pallas as pl
from jax.experimental.pallas import tpu as pltpu
from jax.experimental.pallas import tpu_sc as plsc

_DELTA_VAR = 0.1
_DELTA_DIST = 0.3
_ALPHA = 1.0
_BETA = 1.0

_C = 128
_D = 16
_HW = 128 * 128

_NT = 16
_L = 16
_SPMX = _C * _D + _L


def _tc_body(seg_ref, emb_ref, aff_ref, hflat_ref, spmx_ref):
    seg = seg_ref[...]
    emb = emb_ref[...]
    ids = lax.broadcasted_iota(jnp.int32, (_C, _HW), 0)
    oh = (ids == seg).astype(jnp.float32)
    dn = (((1,), (1,)), ((), ()))
    sp_sums = lax.dot_general(oh, emb, dn, preferred_element_type=jnp.float32)
    n = jnp.sum(oh, axis=1, keepdims=True)
    inv_n = 1.0 / jnp.maximum(n, 1.0)
    means = sp_sums * inv_n
    dn0 = (((0,), (0,)), ((), ()))
    mean_t = lax.dot_general(means, oh, dn0, preferred_element_type=jnp.float32)
    dots = jnp.sum(mean_t * emb, axis=0, keepdims=True)
    t = jnp.clip(1.0 - dots - _DELTA_VAR, 0.0, None)
    seg_t = lax.dot_general(oh, t, dn, preferred_element_type=jnp.float32)
    c_t = (jnp.max(seg) + 1).astype(jnp.float32)
    intra = _BETA * jnp.sum(seg_t * inv_n) / c_t
    spmx_ref[...] = jnp.concatenate(
        [means, jnp.full((1, _D), intra, jnp.float32)], axis=0)

    a0 = aff_ref[0:1, :]
    a1 = aff_ref[1:2, :]
    hraw = 0.5 * (a0 + a1)
    hs = hraw - jnp.min(hraw)
    hflat_ref[...] = hs / (jnp.max(hs) + 1e-6)


def _tc_call(seg_row, emb, aff2):
    return pl.pallas_call(
        _tc_body,
        out_shape=(
            jax.ShapeDtypeStruct((1, _HW), jnp.float32),
            jax.ShapeDtypeStruct((_C + 1, _D), jnp.float32),
        ),
    )(seg_row, emb, aff2)


def _sc_body(P, E, Ppad, Epad,
             hflat_hbm, spmx_hbm, pa_hbm, pb_hbm, peid_hbm, eu_hbm, ev_hbm,
             out_hbm,
             hflat_v, spm_v, pa_v, pb_v, peid_v, vals_v, ones_v,
             eu_v, ev_v, sums_v, cnts_v, part_v, partall_v, out_v,
             sums_sh, cnts_sh, part_sh, sem_in, sem_sc):
    cid = lax.axis_index("c")
    sid = lax.axis_index("s")
    pchunk = Ppad // _NT
    echunk = Epad // _NT
    n_pgrp = pchunk // _L
    n_egrp = echunk // _L
    n_prow = pchunk // 128

    @pl.when(cid == 0)
    def _():
        d_h = pltpu.async_copy(hflat_hbm, hflat_v, sem_in)
        d_m = pltpu.async_copy(spmx_hbm, spm_v, sem_in)
        d_pa = pltpu.async_copy(pa_hbm.at[sid], pa_v, sem_in)
        d_pb = pltpu.async_copy(pb_hbm.at[sid], pb_v, sem_in)
        d_pe = pltpu.async_copy(peid_hbm.at[sid], peid_v, sem_in)
        d_eu = pltpu.async_copy(eu_hbm.at[sid], eu_v, sem_in)
        d_ev = pltpu.async_copy(ev_hbm.at[sid], ev_v, sem_in)

        def zero_body(j, _):
            off = pl.multiple_of(j * _L, _L)
            sums_v[pl.ds(off, _L)] = jnp.zeros((_L,), jnp.float32)
            return 0
        lax.fori_loop(0, echunk // _L, zero_body, 0)
        pltpu.sync_copy(sums_v, sums_sh.at[pl.ds(sid * echunk, echunk)])
        pltpu.sync_copy(sums_v, cnts_sh.at[pl.ds(sid * echunk, echunk)])
        plsc.subcore_barrier()

        d_h.wait(); d_pa.wait(); d_pb.wait(); d_pe.wait()
        base_p = sid * pchunk
        lanes = lax.iota(jnp.int32, _L)

        def pair_body(g, _):
            off = pl.multiple_of(g * _L, _L)
            ia = pa_v[pl.ds(off, _L)]
            ib = pb_v[pl.ds(off, _L)]
            ha = plsc.load_gather(hflat_v, [ia])
            hb = plsc.load_gather(hflat_v, [ib])
            val = 0.5 * (ha + hb)
            gidx = base_p + g * _L + lanes
            m = gidx < P
            q = lax.div(g, 8)
            r = lax.rem(g, 8)
            roff = pl.multiple_of(r * _L, _L)
            vals_v[q, pl.ds(roff, _L)] = jnp.where(m, val, 0.0)
            ones_v[q, pl.ds(roff, _L)] = jnp.where(m, 1.0, 0.0)
            return 0
        lax.fori_loop(0, n_pgrp, pair_body, 0)

        descs = []
        for j in range(n_prow):
            descs.append(pltpu.async_copy(
                vals_v.at[j], sums_sh.at[peid_v.at[j]], sem_sc, add=True))
            descs.append(pltpu.async_copy(
                ones_v.at[j], cnts_sh.at[peid_v.at[j]], sem_sc, add=True))
        for dsc in descs:
            dsc.wait()
        plsc.subcore_barrier()

        base_e = sid * echunk
        d_s = pltpu.async_copy(sums_sh.at[pl.ds(base_e, echunk)], sums_v,
                               sem_in)
        d_c = pltpu.async_copy(cnts_sh.at[pl.ds(base_e, echunk)], cnts_v,
                               sem_in)
        d_m.wait(); d_eu.wait(); d_ev.wait(); d_s.wait(); d_c.wait()

        def edge_body(g, acc):
            off = pl.multiple_of(g * _L, _L)
            u = eu_v[pl.ds(off, _L)]
            v = ev_v[pl.ds(off, _L)]
            s = sums_v[pl.ds(off, _L)]
            c = cnts_v[pl.ds(off, _L)]
            w = s / jnp.maximum(c, 1.0)
            ub = u * _D
            vb = v * _D
            dacc = jnp.zeros((_L,), jnp.float32)
            for dd in range(_D):
                mu = plsc.load_gather(spm_v, [ub + dd])
                mv = plsc.load_gather(spm_v, [vb + dd])
                dacc = dacc + mu * mv
            inter = jnp.clip(_DELTA_DIST - (1.0 - dacc) * w, 0.0, None)
            gidx = base_e + g * _L + lanes
            return acc + jnp.where(gidx < E, inter, 0.0)

        acc = lax.fori_loop(0, n_egrp, edge_body, jnp.zeros((_L,), jnp.float32))
        part_v[...] = acc
        pltpu.sync_copy(part_v, part_sh.at[sid])
        plsc.subcore_barrier()

        @pl.when(sid == 0)
        def _():
            pltpu.sync_copy(part_sh, partall_v)
            tot = jnp.zeros((_L,), jnp.float32)
            for t in range(_NT):
                tot = tot + partall_v[t]
            inter_total = lax.reduce(tot, 0.0, lax.add, (0,))
            intra_vec = spm_v[pl.ds(_C * _D, _L)]
            out_v[...] = jnp.full((_L,), _ALPHA * inter_total * (1.0 / E),
                                  jnp.float32) + intra_vec
            pltpu.sync_copy(out_v, out_hbm)


def _sc_call(P, E, Ppad, Epad, hflat, spmx, pa3, pb3, peid3, eu2, ev2):
    pchunk = Ppad // _NT
    echunk = Epad // _NT
    n_prow = pchunk // 128
    mesh = plsc.VectorSubcoreMesh(core_axis_name="c", subcore_axis_name="s",
                                  num_cores=2, num_subcores=_NT)
    kern = pl.kernel(
        functools.partial(_sc_body, P, E, Ppad, Epad),
        out_type=jax.ShapeDtypeStruct((_L,), jnp.float32),
        mesh=mesh,
        compiler_params=pltpu.CompilerParams(needs_layout_passes=False),
        scratch_types=[
            pltpu.VMEM((_HW,), jnp.float32),
            pltpu.VMEM((_SPMX,), jnp.float32),
            pltpu.VMEM((pchunk,), jnp.int32),
            pltpu.VMEM((pchunk,), jnp.int32),
            pltpu.VMEM((n_prow, 128), jnp.int32),
            pltpu.VMEM((n_prow, 128), jnp.float32),
            pltpu.VMEM((n_prow, 128), jnp.float32),
            pltpu.VMEM((echunk,), jnp.int32),
            pltpu.VMEM((echunk,), jnp.int32),
            pltpu.VMEM((echunk,), jnp.float32),
            pltpu.VMEM((echunk,), jnp.float32),
            pltpu.VMEM((_L,), jnp.float32),
            pltpu.VMEM((_NT, _L), jnp.float32),
            pltpu.VMEM((_L,), jnp.float32),
            pltpu.VMEM_SHARED((Epad,), jnp.float32),
            pltpu.VMEM_SHARED((Epad,), jnp.float32),
            pltpu.VMEM_SHARED((_NT, _L), jnp.float32),
            pltpu.SemaphoreType.DMA,
            pltpu.SemaphoreType.DMA,
        ],
    )
    return kern(hflat, spmx, pa3, pb3, peid3, eu2, ev2)


def _pad_to(x, n):
    return jnp.concatenate([x, jnp.zeros((n - x.shape[0],), x.dtype)])


def kernel(embeddings, sp_seg, affs, offs, edges, pair_edge_ids,
           pair_pix_a, pair_pix_b):
    del offs
    seg_row = sp_seg.reshape(1, _HW).astype(jnp.int32)
    emb = embeddings.reshape(_D, _HW).astype(jnp.float32)
    aff2 = affs[0, :2].reshape(2, _HW).astype(jnp.float32)

    hflat, spmx = _tc_call(seg_row, emb, aff2)

    P = pair_pix_a.shape[0]
    E = edges.shape[1]
    Ppad = -(-P // (_NT * 128)) * (_NT * 128)
    Epad = -(-E // (_NT * 128)) * (_NT * 128)
    pchunk = Ppad // _NT
    echunk = Epad // _NT
    pa3 = _pad_to(pair_pix_a.astype(jnp.int32), Ppad).reshape(_NT, pchunk)
    pb3 = _pad_to(pair_pix_b.astype(jnp.int32), Ppad).reshape(_NT, pchunk)
    peid3 = _pad_to(pair_edge_ids.astype(jnp.int32), Ppad).reshape(
        _NT, pchunk // 128, 128)
    eu2 = _pad_to(edges[0].astype(jnp.int32), Epad).reshape(_NT, echunk)
    ev2 = _pad_to(edges[1].astype(jnp.int32), Epad).reshape(_NT, echunk)

    out_vec = _sc_call(P, E, Ppad, Epad,
                       hflat.reshape(_HW), spmx.reshape(_SPMX),
                       pa3, pb3, peid3, eu2, ev2)
    return out_vec[0]
